# Initial kernel scaffold; baseline (speedup 1.0000x reference)
import jax
import jax.numpy as jnp
from jax.experimental import pallas as pl


def kernel(input_batch, W_router, W_in, W_out):
    raise NotImplementedError("write your pallas kernel here")


# speedup vs baseline: 1.6018x; 1.6018x over previous
import functools

import jax
import jax.numpy as jnp
from jax import lax
from jax.experimental import pallas as pl
from jax.experimental.pallas import tpu as pltpu
from jax.experimental.pallas import tpu_sc as plsc

E = 8
D = 1024
F = 2048
T = 4096

TM = 256
NP = T * 2 + E * TM
NT = NP // TM
SP_LEN = 64

NC = 2
NS = 16
NW = NC * NS
TPW = T // NW
CH_S = 64
CH_C = 32


def _plan_body(tok_ref, wr_ref, dst0_ref, dst1_ref, g0_ref, g1_ref, sp_ref):
    tokens = tok_ref[...]
    logits = jnp.dot(tokens, wr_ref[...], preferred_element_type=jnp.float32)
    probs = jax.nn.softmax(logits, axis=-1)

    eio = lax.broadcasted_iota(jnp.int32, (T, E), 1)
    m1 = jnp.max(probs, axis=-1, keepdims=True)
    idx1 = jnp.min(jnp.where(probs == m1, eio, E), axis=-1, keepdims=True)
    oh0 = (eio == idx1).astype(jnp.float32)
    probs2 = jnp.where(oh0 > 0, -1.0, probs)
    m2 = jnp.max(probs2, axis=-1, keepdims=True)
    idx2 = jnp.min(jnp.where(probs2 == m2, eio, E), axis=-1, keepdims=True)
    oh1 = (eio == idx2).astype(jnp.float32)

    ssum = m1 + m2
    g0_ref[...] = jnp.broadcast_to(m1 / ssum, (T, 16))
    g1_ref[...] = jnp.broadcast_to(m2 / ssum, (T, 16))

    C = 1024
    rio = lax.broadcasted_iota(jnp.int32, (C, C), 0)
    cio = lax.broadcasted_iota(jnp.int32, (C, C), 1)
    tri = (rio > cio).astype(jnp.float32)

    def excl_cumsum_ranks(oh):
        carry = jnp.zeros((1, E), jnp.float32)
        ranks = []
        for c in range(T // C):
            ohc = oh[c * C:(c + 1) * C]
            local = jnp.dot(tri, ohc, preferred_element_type=jnp.float32)
            cum = local + carry
            ranks.append(jnp.sum(cum * ohc, axis=-1, keepdims=True))
            carry = carry + jnp.sum(ohc, axis=0, keepdims=True)
        return jnp.concatenate(ranks, axis=0), carry

    rank0, c0 = excl_cumsum_ranks(oh0)
    rank1, c1 = excl_cumsum_ranks(oh1)
    cnt = c0 + c1
    pcnt = jnp.ceil(cnt / TM) * TM
    uio = lax.broadcasted_iota(jnp.int32, (E, E), 0)
    vio = lax.broadcasted_iota(jnp.int32, (E, E), 1)
    upper = (uio < vio).astype(jnp.float32)
    pstart = jnp.dot(pcnt, upper, preferred_element_type=jnp.float32)

    dst0 = jnp.sum(pstart * oh0, axis=-1, keepdims=True) + rank0
    dst1 = jnp.sum((pstart + c0) * oh1, axis=-1, keepdims=True) + rank1
    dst0_ref[...] = dst0.astype(jnp.int32)
    dst1_ref[...] = dst1.astype(jnp.int32)

    pend = pstart + pcnt
    lio = lax.broadcasted_iota(jnp.int32, (SP_LEN, E), 0).astype(jnp.float32)
    tilebase = (lio - 1.0) * TM
    em = jnp.sum((tilebase >= pend).astype(jnp.float32), axis=-1, keepdims=True)
    em = jnp.minimum(em, float(E - 1))
    nused = jnp.sum(pcnt, axis=-1, keepdims=True) / TM
    l0 = lax.broadcasted_iota(jnp.int32, (SP_LEN, 1), 0)
    sp_ref[...] = jnp.where(l0 == 0, nused, em).astype(jnp.int32)


def _run_plan(tokens, w_router):
    outs = pl.pallas_call(
        _plan_body,
        out_shape=(
            jax.ShapeDtypeStruct((T, 1), jnp.int32),
            jax.ShapeDtypeStruct((T, 1), jnp.int32),
            jax.ShapeDtypeStruct((T, 16), jnp.float32),
            jax.ShapeDtypeStruct((T, 16), jnp.float32),
            jax.ShapeDtypeStruct((SP_LEN, 1), jnp.int32),
        ),
    )(tokens, w_router)
    return outs


def _make_scatter():
    mesh = plsc.VectorSubcoreMesh(core_axis_name="c", subcore_axis_name="s")

    @functools.partial(
        pl.kernel, mesh=mesh,
        out_type=jax.ShapeDtypeStruct((NP, D), jnp.float32),
        scratch_types=[
            pltpu.VMEM((CH_S, D), jnp.float32),
            pltpu.VMEM((CH_S,), jnp.int32),
            pltpu.VMEM((CH_S,), jnp.int32),
            pltpu.SemaphoreType.DMA,
        ],
    )
    def scatter(tok_hbm, dstf_hbm, x_hbm, rows_v, i0_v, i1_v, sem):
        wid = lax.axis_index("s") * NC + lax.axis_index("c")
        for c in range(TPW // CH_S):
            b = wid * TPW + c * CH_S
            pltpu.sync_copy(tok_hbm.at[pl.ds(b, CH_S)], rows_v)
            pltpu.sync_copy(dstf_hbm.at[pl.ds(b, CH_S)], i0_v)
            pltpu.sync_copy(dstf_hbm.at[pl.ds(T + b, CH_S)], i1_v)
            pltpu.async_copy(rows_v, x_hbm.at[i0_v], sem).wait()
            pltpu.async_copy(rows_v, x_hbm.at[i1_v], sem).wait()

    return scatter


def _ffn_body(sp_ref, x_ref, win_ref, wout_ref, y_ref):
    p = pl.program_id(0)

    @pl.when(p < sp_ref[0])
    def _():
        h = jnp.dot(x_ref[...], win_ref[0], preferred_element_type=jnp.float32)
        h = jnp.maximum(h, 0.0)
        y_ref[...] = jnp.dot(h, wout_ref[0], preferred_element_type=jnp.float32)


def _run_ffn(sp, x_sorted, w_in, w_out):
    grid_spec = pltpu.PrefetchScalarGridSpec(
        num_scalar_prefetch=1,
        grid=(NT,),
        in_specs=[
            pl.BlockSpec((TM, D), lambda p, sp: (p, 0)),
            pl.BlockSpec((1, D, F), lambda p, sp: (sp[1 + p], 0, 0)),
            pl.BlockSpec((1, F, D), lambda p, sp: (sp[1 + p], 0, 0)),
        ],
        out_specs=pl.BlockSpec((TM, D), lambda p, sp: (p, 0)),
    )
    return pl.pallas_call(
        _ffn_body,
        grid_spec=grid_spec,
        out_shape=jax.ShapeDtypeStruct((NP, D), jnp.float32),
        compiler_params=pltpu.CompilerParams(
            dimension_semantics=("arbitrary",),
        ),
    )(sp, x_sorted, w_in, w_out)


def _make_combine():
    mesh = plsc.VectorSubcoreMesh(core_axis_name="c", subcore_axis_name="s")

    @functools.partial(
        pl.kernel, mesh=mesh,
        out_type=jax.ShapeDtypeStruct((T, D), jnp.float32),
        scratch_types=[
            pltpu.VMEM((CH_C, D), jnp.float32),
            pltpu.VMEM((CH_C, D), jnp.float32),
            pltpu.VMEM((CH_C,), jnp.int32),
            pltpu.VMEM((CH_C,), jnp.int32),
            pltpu.VMEM((CH_C, 16), jnp.float32),
            pltpu.VMEM((CH_C, 16), jnp.float32),
            pltpu.SemaphoreType.DMA,
        ],
    )
    def combine(y_hbm, dstf_hbm, gf_hbm, out_hbm,
                buf0, buf1, i0_v, i1_v, g0_v, g1_v, sem):
        wid = lax.axis_index("s") * NC + lax.axis_index("c")
        for c in range(TPW // CH_C):
            b = wid * TPW + c * CH_C
            pltpu.sync_copy(dstf_hbm.at[pl.ds(b, CH_C)], i0_v)
            pltpu.sync_copy(dstf_hbm.at[pl.ds(T + b, CH_C)], i1_v)
            pltpu.sync_copy(gf_hbm.at[pl.ds(b, CH_C)], g0_v)
            pltpu.sync_copy(gf_hbm.at[pl.ds(T + b, CH_C)], g1_v)
            cp0 = pltpu.async_copy(y_hbm.at[i0_v], buf0, sem)
            cp1 = pltpu.async_copy(y_hbm.at[i1_v], buf1, sem)
            cp0.wait()
            cp1.wait()

            def row_body(i, carry):
                s0 = g0_v[i, pl.ds(0, 16)]
                s1 = g1_v[i, pl.ds(0, 16)]
                for j in range(D // 16):
                    a = buf0[i, pl.ds(j * 16, 16)]
                    bb = buf1[i, pl.ds(j * 16, 16)]
                    buf0[i, pl.ds(j * 16, 16)] = a * s0 + bb * s1
                return carry

            lax.fori_loop(0, CH_C, row_body, 0)
            pltpu.sync_copy(buf0, out_hbm.at[pl.ds(b, CH_C)])

    return combine


def kernel(input_batch, W_router, W_in, W_out):
    Bb, Ss, Dm = input_batch.shape
    tokens = input_batch.reshape(-1, Dm)

    dst0, dst1, g0, g1, sp = _run_plan(tokens, W_router)
    dstf = jnp.concatenate([dst0.reshape(-1), dst1.reshape(-1)])
    gf = jnp.concatenate([g0, g1])
    sp = sp.reshape(-1)

    x_sorted = _make_scatter()(tokens, dstf)
    y_sorted = _run_ffn(sp, x_sorted, W_in, W_out)
    out = _make_combine()(y_sorted, dstf, gf)
    return out.reshape(Bb, Ss, Dm)
